# Initial kernel scaffold; baseline (speedup 1.0000x reference)
import jax
import jax.numpy as jnp
from jax.experimental import pallas as pl


def kernel(x, edge_index, w1_init, w1, w1_root, b1, w2_init, w2, w2_root, b2):
    raise NotImplementedError("write your pallas kernel here")


# speedup vs baseline: 12.5579x; 12.5579x over previous
import functools

import jax
import jax.numpy as jnp
from jax import lax
from jax.experimental import pallas as pl
from jax.experimental.pallas import tpu as pltpu
from jax.experimental.pallas import tpu_sc as plsc

N = 100000
E = 1600000
F_IN = 128
HID = 16
C_OUT = 40
K = 3

NC, NS = 2, 16
NW = NC * NS
NP = 100096
R_TILE = NP // NS
CH = 1000
CZ = 782
BN = 1088
NBLK = NP // BN

_HIGH = jax.lax.Precision.HIGHEST


def _Z():
    return jnp.int32(0)


def _mesh():
    return plsc.VectorSubcoreMesh(
        core_axis_name="c", subcore_axis_name="s", num_cores=NC, num_subcores=NS
    )


def _make_deg():
    etile = E // NW
    nchunks = etile // CH

    @functools.partial(
        pl.kernel,
        out_type=jax.ShapeDtypeStruct((NC * NP, 16), jnp.float32),
        mesh=_mesh(),
        scratch_types=[
            pltpu.VMEM((CH,), jnp.int32),
            pltpu.VMEM((CH, 16), jnp.float32),
            pltpu.VMEM((CZ, 16), jnp.float32),
            pltpu.VMEM_SHARED((NP, 16), jnp.float32),
        ],
        compiler_params=pltpu.CompilerParams(use_tc_tiling_on_sc=False),
    )
    def deg_kernel(col_hbm, zeros_hbm, ones_hbm, out_hbm, cidx, ones, bounce, acc):
        c = lax.axis_index("c").astype(jnp.int32)
        s = lax.axis_index("s").astype(jnp.int32)
        pltpu.sync_copy(zeros_hbm.at[pl.ds(0, CZ)], bounce)
        for j in range(R_TILE // CZ):
            pltpu.sync_copy(bounce, acc.at[pl.ds(s * R_TILE + j * CZ, CZ)])
        pltpu.sync_copy(ones_hbm, ones)
        plsc.subcore_barrier()

        def body(i, carry):
            eoff = (c * NS + s) * etile + i * CH
            pltpu.sync_copy(col_hbm.at[pl.ds(eoff, CH)], cidx)
            pltpu.sync_copy(ones, acc.at[cidx], add=True)
            return carry

        lax.fori_loop(jnp.int32(0), jnp.int32(nchunks), body, jnp.int32(0))
        plsc.subcore_barrier()
        for j in range(R_TILE // CZ):
            pltpu.sync_copy(acc.at[pl.ds(s * R_TILE + j * CZ, CZ)], bounce)
            pltpu.sync_copy(bounce,
                            out_hbm.at[pl.ds(c * NP + s * R_TILE + j * CZ, CZ)])

    return deg_kernel


def _make_prop():
    epc = E // NC
    etile = epc // NS
    nchunks = etile // CH

    @functools.partial(
        pl.kernel,
        out_type=jax.ShapeDtypeStruct((NC * K * NP, 16), jnp.float32),
        mesh=_mesh(),
        scratch_types=[
            pltpu.VMEM((CH,), jnp.int32),
            pltpu.VMEM((CH,), jnp.int32),
            pltpu.VMEM((CH, 16), jnp.float32),
            pltpu.VMEM_SHARED((NP, 16), jnp.float32),
            pltpu.SemaphoreType.DMA,
        ],
        compiler_params=pltpu.CompilerParams(use_tc_tiling_on_sc=False),
    )
    def prop_kernel(rowadj_hbm, col_hbm, zeros_hbm, table_hbm, out_hbm,
                    ridx, cidx, rows, acc, gsem):
        c = lax.axis_index("c").astype(jnp.int32)
        s = lax.axis_index("s").astype(jnp.int32)
        for k in range(K):
            pltpu.sync_copy(zeros_hbm.at[pl.ds(0, CZ)], rows.at[pl.ds(0, CZ)])
            for j in range(R_TILE // CZ):
                pltpu.sync_copy(rows.at[pl.ds(0, CZ)],
                                acc.at[pl.ds(s * R_TILE + j * CZ, CZ)])
            plsc.subcore_barrier()

            def body(i, carry):
                eoff = c * epc + s * etile + i * CH
                pltpu.sync_copy(rowadj_hbm.at[pl.ds(k * E + eoff, CH)], ridx)
                pltpu.sync_copy(col_hbm.at[pl.ds(eoff, CH)], cidx)
                pltpu.async_copy(table_hbm.at[ridx], rows, gsem).wait()
                pltpu.sync_copy(rows, acc.at[cidx], add=True)
                return carry

            lax.fori_loop(jnp.int32(0), jnp.int32(nchunks), body, jnp.int32(0))
            plsc.subcore_barrier()
            t = c * K + k
            for j in range(R_TILE // CZ):
                pltpu.sync_copy(acc.at[pl.ds(s * R_TILE + j * CZ, CZ)],
                                rows.at[pl.ds(0, CZ)])
                pltpu.sync_copy(rows.at[pl.ds(0, CZ)],
                                out_hbm.at[pl.ds(t * NP + s * R_TILE + j * CZ, CZ)])

    return prop_kernel


_deg_call = _make_deg()
_prop = _make_prop()


def _dis_of(degp_blk):
    deg = (degp_blk[0] + degp_blk[1])[:, :1]
    return jnp.where(deg > 0, lax.rsqrt(jnp.where(deg > 0, deg, 1.0)), 0.0)


def _sum48(p_ref):
    sums = p_ref[0] + p_ref[1]
    return jnp.concatenate([sums[k] for k in range(K)], axis=1)


def _slice_specs():
    return [
        pl.BlockSpec((NC, K, BN, HID), lambda n: (_Z(), _Z(), n, _Z())),
        pl.BlockSpec((2, BN, 16), lambda n: (_Z(), n, _Z())),
    ]


def _stage_a(x_pad, degp, wa, b1c):
    def body(x_ref, degp_ref, wa_ref, b1_ref, t0_ref, r1_ref):
        dis = _dis_of(degp_ref[...])
        h = jnp.dot(x_ref[...], wa_ref[...], precision=_HIGH,
                    preferred_element_type=jnp.float32)
        r1_ref[...] = h[:, 3 * HID:] + b1_ref[...]
        t = dis * h[:, : 3 * HID]
        for k in range(K):
            t0_ref[k] = t[:, HID * k:HID * (k + 1)]

    return pl.pallas_call(
        body,
        grid=(NBLK,),
        in_specs=[
            pl.BlockSpec((BN, F_IN), lambda n: (n, _Z())),
            pl.BlockSpec((2, BN, 16), lambda n: (_Z(), n, _Z())),
            pl.BlockSpec((F_IN, 6 * HID), lambda n: (_Z(), _Z())),
            pl.BlockSpec((1, 3 * HID), lambda n: (_Z(), _Z())),
        ],
        out_specs=[
            pl.BlockSpec((K, BN, HID), lambda n: (_Z(), n, _Z())),
            pl.BlockSpec((BN, 3 * HID), lambda n: (n, _Z())),
        ],
        out_shape=[
            jax.ShapeDtypeStruct((K, NP, HID), jnp.float32),
            jax.ShapeDtypeStruct((NP, 3 * HID), jnp.float32),
        ],
    )(x_pad, degp, wa, b1c)


def _stage_b(p1, degp, r1, w1bd):
    def body(p_ref, degp_ref, r1_ref, w_ref, t1_ref):
        dis = _dis_of(degp_ref[...])
        o = jax.nn.relu(dis * _sum48(p_ref) + r1_ref[...])
        t1 = dis * jnp.dot(o, w_ref[...], precision=_HIGH,
                           preferred_element_type=jnp.float32)
        for k in range(K):
            t1_ref[k] = t1[:, HID * k:HID * (k + 1)]

    return pl.pallas_call(
        body,
        grid=(NBLK,),
        in_specs=_slice_specs() + [
            pl.BlockSpec((BN, 3 * HID), lambda n: (n, _Z())),
            pl.BlockSpec((3 * HID, 3 * HID), lambda n: (_Z(), _Z())),
        ],
        out_specs=pl.BlockSpec((K, BN, HID), lambda n: (_Z(), n, _Z())),
        out_shape=jax.ShapeDtypeStruct((K, NP, HID), jnp.float32),
    )(p1, degp, r1, w1bd)


def _slices9(t_ref, vals, dis):
    zpad = jnp.zeros_like(vals[:, :8])
    for k in range(K):
        zk = jnp.concatenate([dis * vals[:, C_OUT * k:C_OUT * (k + 1)], zpad],
                             axis=1)
        for j in range(3):
            t_ref[3 * k + j] = zk[:, HID * j:HID * (j + 1)]


def _stage_c(p2, degp, r1, wc, b2c):
    def body(p_ref, degp_ref, r1_ref, wc_ref, b2_ref, t2_ref, r2_ref):
        dis = _dis_of(degp_ref[...])
        o = jax.nn.relu(dis * _sum48(p_ref) + r1_ref[...])
        h = (o[:, :HID] + o[:, HID:2 * HID] + o[:, 2 * HID:]) * (1.0 / 3.0)
        h = jax.nn.relu(h)
        z = jnp.dot(h, wc_ref[...], precision=_HIGH,
                    preferred_element_type=jnp.float32)
        r2_ref[...] = z[:, 3 * C_OUT:] + b2_ref[...]
        _slices9(t2_ref, z[:, : 3 * C_OUT], dis)

    return pl.pallas_call(
        body,
        grid=(NBLK,),
        in_specs=_slice_specs() + [
            pl.BlockSpec((BN, 3 * HID), lambda n: (n, _Z())),
            pl.BlockSpec((HID, 6 * C_OUT), lambda n: (_Z(), _Z())),
            pl.BlockSpec((1, 3 * C_OUT), lambda n: (_Z(), _Z())),
        ],
        out_specs=[
            pl.BlockSpec((3 * K, BN, HID), lambda n: (_Z(), n, _Z())),
            pl.BlockSpec((BN, 3 * C_OUT), lambda n: (n, _Z())),
        ],
        out_shape=[
            jax.ShapeDtypeStruct((3 * K, NP, HID), jnp.float32),
            jax.ShapeDtypeStruct((NP, 3 * C_OUT), jnp.float32),
        ],
    )(p2, degp, r1, wc, b2c)


def _cat120(p_refs, dis, r2):
    parts = []
    for k in range(K):
        s48 = _sum48(p_refs[k])
        parts.append(s48[:, :C_OUT])
    return dis * jnp.concatenate(parts, axis=1) + r2


def _stage_d(p3s, degp, r2, w2bd):
    def body(pa_ref, pb_ref, pc_ref, degp_ref, r2_ref, w_ref, t3_ref):
        dis = _dis_of(degp_ref[...])
        o = _cat120([pa_ref, pb_ref, pc_ref], dis, r2_ref[...])
        t3 = jnp.dot(o, w_ref[...], precision=_HIGH,
                     preferred_element_type=jnp.float32)
        _slices9(t3_ref, t3, dis)

    return pl.pallas_call(
        body,
        grid=(NBLK,),
        in_specs=[
            pl.BlockSpec((NC, K, BN, HID), lambda n: (_Z(), _Z(), n, _Z())),
            pl.BlockSpec((NC, K, BN, HID), lambda n: (_Z(), _Z(), n, _Z())),
            pl.BlockSpec((NC, K, BN, HID), lambda n: (_Z(), _Z(), n, _Z())),
            pl.BlockSpec((2, BN, 16), lambda n: (_Z(), n, _Z())),
            pl.BlockSpec((BN, 3 * C_OUT), lambda n: (n, _Z())),
            pl.BlockSpec((3 * C_OUT, 3 * C_OUT), lambda n: (_Z(), _Z())),
        ],
        out_specs=pl.BlockSpec((3 * K, BN, HID), lambda n: (_Z(), n, _Z())),
        out_shape=jax.ShapeDtypeStruct((3 * K, NP, HID), jnp.float32),
    )(p3s[0], p3s[1], p3s[2], degp, r2, w2bd)


def _stage_e(p4s, degp, r2):
    def body(pa_ref, pb_ref, pc_ref, degp_ref, r2_ref, out_ref):
        dis = _dis_of(degp_ref[...])
        o = _cat120([pa_ref, pb_ref, pc_ref], dis, r2_ref[...])
        m = (o[:, :C_OUT] + o[:, C_OUT:2 * C_OUT] + o[:, 2 * C_OUT:]) * (1.0 / 3.0)
        mx = jnp.max(m, axis=1, keepdims=True)
        lse = jnp.log(jnp.sum(jnp.exp(m - mx), axis=1, keepdims=True)) + mx
        out_ref[...] = m - lse

    return pl.pallas_call(
        body,
        grid=(NBLK,),
        in_specs=[
            pl.BlockSpec((NC, K, BN, HID), lambda n: (_Z(), _Z(), n, _Z())),
            pl.BlockSpec((NC, K, BN, HID), lambda n: (_Z(), _Z(), n, _Z())),
            pl.BlockSpec((NC, K, BN, HID), lambda n: (_Z(), _Z(), n, _Z())),
            pl.BlockSpec((2, BN, 16), lambda n: (_Z(), n, _Z())),
            pl.BlockSpec((BN, 3 * C_OUT), lambda n: (n, _Z())),
        ],
        out_specs=pl.BlockSpec((BN, C_OUT), lambda n: (n, _Z())),
        out_shape=jax.ShapeDtypeStruct((NP, C_OUT), jnp.float32),
    )(p4s[0], p4s[1], p4s[2], degp, r2)


def kernel(x, edge_index, w1_init, w1, w1_root, b1, w2_init, w2, w2_root, b2):
    f32 = jnp.float32
    row = edge_index[0].astype(jnp.int32)
    col = edge_index[1].astype(jnp.int32)
    x_pad = jnp.pad(x.astype(f32), ((0, NP - N), (0, 0)))

    row_adj = (row[None, :]
               + (jnp.arange(K, dtype=jnp.int32) * NP)[:, None]).reshape(-1)

    zeros16 = jnp.zeros((NP, 16), f32)
    ones16 = jnp.ones((CH, 16), f32)

    wa = jnp.concatenate(
        [jnp.concatenate([w1_init[k] for k in range(K)], axis=1),
         jnp.concatenate([w1_root[k] for k in range(K)], axis=1)], axis=1)
    b1c = b1.astype(f32).reshape(1, K * HID)
    w1bd = jax.scipy.linalg.block_diag(*[w1[k] for k in range(K)]).astype(f32)
    wc = jnp.concatenate(
        [jnp.concatenate([w2_init[k] for k in range(K)], axis=1),
         jnp.concatenate([w2_root[k] for k in range(K)], axis=1)], axis=1)
    b2c = b2.astype(f32).reshape(1, K * C_OUT)
    w2bd = jax.scipy.linalg.block_diag(*[w2[k] for k in range(K)]).astype(f32)

    degp = _deg_call(col, zeros16, ones16).reshape(NC, NP, 16)
    t0, r1 = _stage_a(x_pad, degp, wa, b1c)
    p1 = _prop(row_adj, col, zeros16, t0.reshape(K * NP, HID))
    t1 = _stage_b(p1.reshape(NC, K, NP, HID), degp, r1, w1bd)
    p2 = _prop(row_adj, col, zeros16, t1.reshape(K * NP, HID))
    t2, r2 = _stage_c(p2.reshape(NC, K, NP, HID), degp, r1, wc, b2c)
    p3s = [_prop(row_adj, col, zeros16,
                 t2[3 * k:3 * (k + 1)].reshape(K * NP, HID)
                 ).reshape(NC, K, NP, HID) for k in range(K)]
    t3 = _stage_d(p3s, degp, r2, w2bd)
    p4s = [_prop(row_adj, col, zeros16,
                 t3[3 * k:3 * (k + 1)].reshape(K * NP, HID)
                 ).reshape(NC, K, NP, HID) for k in range(K)]
    out = _stage_e(p4s, degp, r2)
    return out[:N]


# speedup vs baseline: 14.0518x; 1.1190x over previous
import functools

import jax
import jax.numpy as jnp
from jax import lax
from jax.experimental import pallas as pl
from jax.experimental.pallas import tpu as pltpu
from jax.experimental.pallas import tpu_sc as plsc

N = 100000
E = 1600000
F_IN = 128
HID = 16
C_OUT = 40
K = 3

NC, NS = 2, 16
NW = NC * NS
NP = 100096
R_TILE = NP // NS
CH = 400
ECH = E // CH
CPC = ECH // NC
CPT = CPC // NS
CZ = 782
CH_DEG = 1000
BN = 1088
NBLK = NP // BN

_HIGH = jax.lax.Precision.HIGHEST


def _Z():
    return jnp.int32(0)


def _mesh():
    return plsc.VectorSubcoreMesh(
        core_axis_name="c", subcore_axis_name="s", num_cores=NC, num_subcores=NS
    )


def _make_deg():
    etile = E // NW
    nchunks = etile // CH_DEG

    @functools.partial(
        pl.kernel,
        out_type=jax.ShapeDtypeStruct((NC * NP, 16), jnp.float32),
        mesh=_mesh(),
        scratch_types=[
            pltpu.VMEM((CH_DEG,), jnp.int32),
            pltpu.VMEM((CH_DEG, 16), jnp.float32),
            pltpu.VMEM((CZ, 16), jnp.float32),
            pltpu.VMEM_SHARED((NP, 16), jnp.float32),
        ],
        compiler_params=pltpu.CompilerParams(use_tc_tiling_on_sc=False),
    )
    def deg_kernel(col_hbm, zeros_hbm, ones_hbm, out_hbm, cidx, ones, bounce, acc):
        c = lax.axis_index("c").astype(jnp.int32)
        s = lax.axis_index("s").astype(jnp.int32)
        pltpu.sync_copy(zeros_hbm.at[pl.ds(0, CZ)], bounce)
        for j in range(R_TILE // CZ):
            pltpu.sync_copy(bounce, acc.at[pl.ds(s * R_TILE + j * CZ, CZ)])
        pltpu.sync_copy(ones_hbm, ones)
        plsc.subcore_barrier()

        def body(i, carry):
            eoff = (c * NS + s) * etile + i * CH_DEG
            pltpu.sync_copy(col_hbm.at[pl.ds(eoff, CH_DEG)], cidx)
            pltpu.sync_copy(ones, acc.at[cidx], add=True)
            return carry

        lax.fori_loop(jnp.int32(0), jnp.int32(nchunks), body, jnp.int32(0))
        plsc.subcore_barrier()
        for j in range(R_TILE // CZ):
            pltpu.sync_copy(acc.at[pl.ds(s * R_TILE + j * CZ, CZ)], bounce)
            pltpu.sync_copy(bounce,
                            out_hbm.at[pl.ds(c * NP + s * R_TILE + j * CZ, CZ)])

    return deg_kernel


def _make_prop(nslots):
    @functools.partial(
        pl.kernel,
        out_type=jax.ShapeDtypeStruct((NC * nslots * NP, 16), jnp.float32),
        mesh=_mesh(),
        scratch_types=[
            pltpu.VMEM((3, CH), jnp.int32),
            pltpu.VMEM((3, CH), jnp.int32),
            pltpu.VMEM((2 * CH, 16), jnp.float32),
            pltpu.VMEM((CZ, 16), jnp.float32),
            pltpu.VMEM_SHARED((NP, 16), jnp.float32),
            pltpu.SemaphoreType.DMA,
            pltpu.SemaphoreType.DMA,
            pltpu.SemaphoreType.DMA,
        ],
        compiler_params=pltpu.CompilerParams(use_tc_tiling_on_sc=False),
    )
    def prop_kernel(row2d, col2d, zeros_hbm, *refs):
        tables = refs[:nslots]
        out_hbm = refs[nslots]
        ridx, cidx, rows, bounce, acc, isem, gsem, ssem = refs[nslots + 1:]
        c = lax.axis_index("c").astype(jnp.int32)
        s = lax.axis_index("s").astype(jnp.int32)
        crow0 = c * CPC + s * CPT

        def idx_issue(j, r):
            pltpu.async_copy(col2d.at[crow0 + j], cidx.at[r], isem)
            pltpu.async_copy(row2d.at[crow0 + j], ridx.at[r], isem)

        def idx_drain(r):
            pltpu.make_async_copy(col2d.at[_Z()], cidx.at[r], isem).wait()
            pltpu.make_async_copy(row2d.at[_Z()], ridx.at[r], isem).wait()

        for k in range(nslots):
            tbl = tables[k]

            def gather_issue(j, p):
                pltpu.async_copy(tbl.at[ridx.at[lax.rem(j, jnp.int32(3))]],
                                 rows.at[pl.ds(p * CH, CH)], gsem)

            def gather_drain(p):
                pltpu.make_async_copy(tbl.at[ridx.at[_Z()]],
                                      rows.at[pl.ds(p * CH, CH)], gsem).wait()

            def scatter_issue(j, p):
                pltpu.async_copy(rows.at[pl.ds(p * CH, CH)],
                                 acc.at[cidx.at[lax.rem(j, jnp.int32(3))]], ssem, add=True)

            def scatter_drain():
                pltpu.make_async_copy(rows.at[pl.ds(0, CH)],
                                      acc.at[cidx.at[_Z()]], ssem).wait()

            pltpu.sync_copy(zeros_hbm.at[pl.ds(0, CZ)], bounce)
            for j in range(R_TILE // CZ):
                pltpu.async_copy(bounce,
                                 acc.at[pl.ds(s * R_TILE + j * CZ, CZ)], gsem)
            for j in range(R_TILE // CZ):
                pltpu.make_async_copy(bounce,
                                      acc.at[pl.ds(s * R_TILE + j * CZ, CZ)],
                                      gsem).wait()
            plsc.subcore_barrier()

            idx_issue(jnp.int32(0), jnp.int32(0))
            idx_issue(jnp.int32(1), jnp.int32(1))
            idx_drain(jnp.int32(0))
            gather_issue(jnp.int32(0), jnp.int32(0))

            def body(j, carry):
                p = jnp.bitwise_and(j, 1)
                gather_drain(p)
                scatter_issue(j, p)

                @pl.when(j + 1 < CPT)
                def _():
                    idx_drain(lax.rem(j + 1, jnp.int32(3)))

                    @pl.when(j >= 1)
                    def _():
                        scatter_drain()

                    gather_issue(j + 1, 1 - p)

                @pl.when(j + 2 < CPT)
                def _():
                    idx_issue(j + 2, lax.rem(j + 2, jnp.int32(3)))

                return carry

            lax.fori_loop(jnp.int32(0), jnp.int32(CPT), body, jnp.int32(0))
            scatter_drain()
            scatter_drain()
            plsc.subcore_barrier()

            t = c * nslots + k
            for j in range(R_TILE // CZ):
                pltpu.sync_copy(acc.at[pl.ds(s * R_TILE + j * CZ, CZ)], bounce)
                pltpu.sync_copy(bounce,
                                out_hbm.at[pl.ds(t * NP + s * R_TILE + j * CZ, CZ)])

    return prop_kernel


_deg_call = _make_deg()
_prop3 = _make_prop(3)
_prop9 = _make_prop(9)


def _dis_of(degp_blk):
    deg = (degp_blk[0] + degp_blk[1])[:, :1]
    return jnp.where(deg > 0, lax.rsqrt(jnp.where(deg > 0, deg, 1.0)), 0.0)


def _sum48(p_ref):
    sums = p_ref[0] + p_ref[1]
    return jnp.concatenate([sums[k] for k in range(K)], axis=1)


def _stage_a(x_pad, degp, wa, b1c):
    def body(x_ref, degp_ref, wa_ref, b1_ref, ta_ref, tb_ref, tc_ref, r1_ref):
        dis = _dis_of(degp_ref[...])
        h = jnp.dot(x_ref[...], wa_ref[...], precision=_HIGH,
                    preferred_element_type=jnp.float32)
        r1_ref[...] = h[:, 3 * HID:] + b1_ref[...]
        t = dis * h[:, : 3 * HID]
        for k, ref in enumerate((ta_ref, tb_ref, tc_ref)):
            ref[...] = t[:, HID * k:HID * (k + 1)]

    slot = pl.BlockSpec((BN, HID), lambda n: (n, _Z()))
    return pl.pallas_call(
        body,
        grid=(NBLK,),
        in_specs=[
            pl.BlockSpec((BN, F_IN), lambda n: (n, _Z())),
            pl.BlockSpec((2, BN, 16), lambda n: (_Z(), n, _Z())),
            pl.BlockSpec((F_IN, 6 * HID), lambda n: (_Z(), _Z())),
            pl.BlockSpec((1, 3 * HID), lambda n: (_Z(), _Z())),
        ],
        out_specs=[slot, slot, slot,
                   pl.BlockSpec((BN, 3 * HID), lambda n: (n, _Z()))],
        out_shape=[jax.ShapeDtypeStruct((NP, HID), jnp.float32)] * 3
        + [jax.ShapeDtypeStruct((NP, 3 * HID), jnp.float32)],
    )(x_pad, degp, wa, b1c)


def _stage_b(p1, degp, r1, w1bd):
    def body(p_ref, degp_ref, r1_ref, w_ref, ta_ref, tb_ref, tc_ref):
        dis = _dis_of(degp_ref[...])
        o = jax.nn.relu(dis * _sum48(p_ref) + r1_ref[...])
        t1 = dis * jnp.dot(o, w_ref[...], precision=_HIGH,
                           preferred_element_type=jnp.float32)
        for k, ref in enumerate((ta_ref, tb_ref, tc_ref)):
            ref[...] = t1[:, HID * k:HID * (k + 1)]

    slot = pl.BlockSpec((BN, HID), lambda n: (n, _Z()))
    return pl.pallas_call(
        body,
        grid=(NBLK,),
        in_specs=[
            pl.BlockSpec((NC, K, BN, HID), lambda n: (_Z(), _Z(), n, _Z())),
            pl.BlockSpec((2, BN, 16), lambda n: (_Z(), n, _Z())),
            pl.BlockSpec((BN, 3 * HID), lambda n: (n, _Z())),
            pl.BlockSpec((3 * HID, 3 * HID), lambda n: (_Z(), _Z())),
        ],
        out_specs=[slot] * 3,
        out_shape=[jax.ShapeDtypeStruct((NP, HID), jnp.float32)] * 3,
    )(p1, degp, r1, w1bd)


def _slices9(t_refs, vals, dis):
    zpad = jnp.zeros_like(vals[:, :8])
    for k in range(K):
        zk = jnp.concatenate([dis * vals[:, C_OUT * k:C_OUT * (k + 1)], zpad],
                             axis=1)
        for j in range(3):
            t_refs[3 * k + j][...] = zk[:, HID * j:HID * (j + 1)]


def _stage_c(p2, degp, r1, wc, b2c):
    def body(p_ref, degp_ref, r1_ref, wc_ref, b2_ref, *out_refs):
        dis = _dis_of(degp_ref[...])
        o = jax.nn.relu(dis * _sum48(p_ref) + r1_ref[...])
        h = (o[:, :HID] + o[:, HID:2 * HID] + o[:, 2 * HID:]) * (1.0 / 3.0)
        h = jax.nn.relu(h)
        z = jnp.dot(h, wc_ref[...], precision=_HIGH,
                    preferred_element_type=jnp.float32)
        out_refs[9][...] = z[:, 3 * C_OUT:] + b2_ref[...]
        _slices9(out_refs[:9], z[:, : 3 * C_OUT], dis)

    slot = pl.BlockSpec((BN, HID), lambda n: (n, _Z()))
    return pl.pallas_call(
        body,
        grid=(NBLK,),
        in_specs=[
            pl.BlockSpec((NC, K, BN, HID), lambda n: (_Z(), _Z(), n, _Z())),
            pl.BlockSpec((2, BN, 16), lambda n: (_Z(), n, _Z())),
            pl.BlockSpec((BN, 3 * HID), lambda n: (n, _Z())),
            pl.BlockSpec((HID, 6 * C_OUT), lambda n: (_Z(), _Z())),
            pl.BlockSpec((1, 3 * C_OUT), lambda n: (_Z(), _Z())),
        ],
        out_specs=[slot] * 9 + [pl.BlockSpec((BN, 3 * C_OUT), lambda n: (n, _Z()))],
        out_shape=[jax.ShapeDtypeStruct((NP, HID), jnp.float32)] * 9
        + [jax.ShapeDtypeStruct((NP, 3 * C_OUT), jnp.float32)],
    )(p2, degp, r1, wc, b2c)


def _cat120(p_ref, dis, r2):
    parts = []
    for k in range(K):
        s48 = jnp.concatenate(
            [p_ref[0, 3 * k + j] + p_ref[1, 3 * k + j] for j in range(3)], axis=1)
        parts.append(s48[:, :C_OUT])
    return dis * jnp.concatenate(parts, axis=1) + r2


def _stage_d(p3, degp, r2, w2bd):
    def body(p_ref, degp_ref, r2_ref, w_ref, *out_refs):
        dis = _dis_of(degp_ref[...])
        o = _cat120(p_ref[...], dis, r2_ref[...])
        t3 = jnp.dot(o, w_ref[...], precision=_HIGH,
                     preferred_element_type=jnp.float32)
        _slices9(out_refs, t3, dis)

    slot = pl.BlockSpec((BN, HID), lambda n: (n, _Z()))
    return pl.pallas_call(
        body,
        grid=(NBLK,),
        in_specs=[
            pl.BlockSpec((NC, 9, BN, HID), lambda n: (_Z(), _Z(), n, _Z())),
            pl.BlockSpec((2, BN, 16), lambda n: (_Z(), n, _Z())),
            pl.BlockSpec((BN, 3 * C_OUT), lambda n: (n, _Z())),
            pl.BlockSpec((3 * C_OUT, 3 * C_OUT), lambda n: (_Z(), _Z())),
        ],
        out_specs=[slot] * 9,
        out_shape=[jax.ShapeDtypeStruct((NP, HID), jnp.float32)] * 9,
    )(p3, degp, r2, w2bd)


def _stage_e(p4, degp, r2):
    def body(p_ref, degp_ref, r2_ref, out_ref):
        dis = _dis_of(degp_ref[...])
        o = _cat120(p_ref[...], dis, r2_ref[...])
        m = (o[:, :C_OUT] + o[:, C_OUT:2 * C_OUT] + o[:, 2 * C_OUT:]) * (1.0 / 3.0)
        mx = jnp.max(m, axis=1, keepdims=True)
        lse = jnp.log(jnp.sum(jnp.exp(m - mx), axis=1, keepdims=True)) + mx
        out_ref[...] = m - lse

    return pl.pallas_call(
        body,
        grid=(NBLK,),
        in_specs=[
            pl.BlockSpec((NC, 9, BN, HID), lambda n: (_Z(), _Z(), n, _Z())),
            pl.BlockSpec((2, BN, 16), lambda n: (_Z(), n, _Z())),
            pl.BlockSpec((BN, 3 * C_OUT), lambda n: (n, _Z())),
        ],
        out_specs=pl.BlockSpec((BN, C_OUT), lambda n: (n, _Z())),
        out_shape=jax.ShapeDtypeStruct((NP, C_OUT), jnp.float32),
    )(p4, degp, r2)


def kernel(x, edge_index, w1_init, w1, w1_root, b1, w2_init, w2, w2_root, b2):
    f32 = jnp.float32
    row = edge_index[0].astype(jnp.int32)
    col = edge_index[1].astype(jnp.int32)
    row2d = row.reshape(ECH, CH)
    col2d = col.reshape(ECH, CH)
    x_pad = jnp.pad(x.astype(f32), ((0, NP - N), (0, 0)))

    zeros16 = jnp.zeros((NP, 16), f32)
    ones16 = jnp.ones((CH_DEG, 16), f32)

    wa = jnp.concatenate(
        [jnp.concatenate([w1_init[k] for k in range(K)], axis=1),
         jnp.concatenate([w1_root[k] for k in range(K)], axis=1)], axis=1)
    b1c = b1.astype(f32).reshape(1, K * HID)
    w1bd = jax.scipy.linalg.block_diag(*[w1[k] for k in range(K)]).astype(f32)
    wc = jnp.concatenate(
        [jnp.concatenate([w2_init[k] for k in range(K)], axis=1),
         jnp.concatenate([w2_root[k] for k in range(K)], axis=1)], axis=1)
    b2c = b2.astype(f32).reshape(1, K * C_OUT)
    w2bd = jax.scipy.linalg.block_diag(*[w2[k] for k in range(K)]).astype(f32)

    degp = _deg_call(col, zeros16, ones16).reshape(NC, NP, 16)
    *t0s, r1 = _stage_a(x_pad, degp, wa, b1c)
    p1 = _prop3(row2d, col2d, zeros16, *t0s).reshape(NC, K, NP, HID)
    t1s = _stage_b(p1, degp, r1, w1bd)
    p2 = _prop3(row2d, col2d, zeros16, *t1s).reshape(NC, K, NP, HID)
    *t2s, r2 = _stage_c(p2, degp, r1, wc, b2c)
    p3 = _prop9(row2d, col2d, zeros16, *t2s).reshape(NC, 9, NP, HID)
    t3s = _stage_d(p3, degp, r2, w2bd)
    p4 = _prop9(row2d, col2d, zeros16, *t3s).reshape(NC, 9, NP, HID)
    out = _stage_e(p4, degp, r2)
    return out[:N]


# speedup vs baseline: 14.1981x; 1.0104x over previous
import functools

import jax
import jax.numpy as jnp
from jax import lax
from jax.experimental import pallas as pl
from jax.experimental.pallas import tpu as pltpu
from jax.experimental.pallas import tpu_sc as plsc

N = 100000
E = 1600000
F_IN = 128
HID = 16
C_OUT = 40
K = 3

NC, NS = 2, 16
NW = NC * NS
NP = 100096
R_TILE = NP // NS
CH = 400
ECH = E // CH
CPC = ECH // NC
CPT = CPC // NS
CZ = 782
CH_DEG = 1000
BN = 1088
NBLK = NP // BN

_HIGH = jax.lax.Precision.HIGHEST


def _Z():
    return jnp.int32(0)


def _mesh():
    return plsc.VectorSubcoreMesh(
        core_axis_name="c", subcore_axis_name="s", num_cores=NC, num_subcores=NS
    )


def _make_deg():
    etile = E // NW
    nchunks = etile // CH_DEG

    @functools.partial(
        pl.kernel,
        out_type=jax.ShapeDtypeStruct((NC * NP, 16), jnp.float32),
        mesh=_mesh(),
        scratch_types=[
            pltpu.VMEM((CH_DEG,), jnp.int32),
            pltpu.VMEM((CH_DEG, 16), jnp.float32),
            pltpu.VMEM((CZ, 16), jnp.float32),
            pltpu.VMEM_SHARED((NP, 16), jnp.float32),
        ],
        compiler_params=pltpu.CompilerParams(use_tc_tiling_on_sc=False),
    )
    def deg_kernel(col_hbm, zeros_hbm, ones_hbm, out_hbm, cidx, ones, bounce, acc):
        c = lax.axis_index("c").astype(jnp.int32)
        s = lax.axis_index("s").astype(jnp.int32)
        pltpu.sync_copy(zeros_hbm.at[pl.ds(0, CZ)], bounce)
        for j in range(R_TILE // CZ):
            pltpu.sync_copy(bounce, acc.at[pl.ds(s * R_TILE + j * CZ, CZ)])
        pltpu.sync_copy(ones_hbm, ones)
        plsc.subcore_barrier()

        def body(i, carry):
            eoff = (c * NS + s) * etile + i * CH_DEG
            pltpu.sync_copy(col_hbm.at[pl.ds(eoff, CH_DEG)], cidx)
            pltpu.sync_copy(ones, acc.at[cidx], add=True)
            return carry

        lax.fori_loop(jnp.int32(0), jnp.int32(nchunks), body, jnp.int32(0))
        plsc.subcore_barrier()
        for j in range(R_TILE // CZ):
            pltpu.sync_copy(acc.at[pl.ds(s * R_TILE + j * CZ, CZ)], bounce)
            pltpu.sync_copy(bounce,
                            out_hbm.at[pl.ds(c * NP + s * R_TILE + j * CZ, CZ)])

    return deg_kernel


def _make_prop(nslots):
    @functools.partial(
        pl.kernel,
        out_type=jax.ShapeDtypeStruct((NC * nslots * NP, 16), jnp.float32),
        mesh=_mesh(),
        scratch_types=[
            pltpu.VMEM((4, CH), jnp.int32),
            pltpu.VMEM((4, CH), jnp.int32),
            pltpu.VMEM((3 * CH, 16), jnp.float32),
            pltpu.VMEM_SHARED((NP, 16), jnp.float32),
            pltpu.SemaphoreType.DMA,
            pltpu.SemaphoreType.DMA,
            pltpu.SemaphoreType.DMA,
        ],
        compiler_params=pltpu.CompilerParams(use_tc_tiling_on_sc=False),
    )
    def prop_kernel(row2d, col2d, zeros_hbm, *refs):
        tables = refs[:nslots]
        out_hbm = refs[nslots]
        ridx, cidx, rows, acc, isem, gsem, ssem = refs[nslots + 1:]
        c = lax.axis_index("c").astype(jnp.int32)
        s = lax.axis_index("s").astype(jnp.int32)
        crow0 = c * CPC + s * CPT
        i3, i4 = jnp.int32(3), jnp.int32(4)

        def idx_issue(j):
            r = lax.rem(j, i4)
            pltpu.async_copy(col2d.at[crow0 + j], cidx.at[r], isem)
            pltpu.async_copy(row2d.at[crow0 + j], ridx.at[r], isem)

        def idx_drain(r):
            pltpu.make_async_copy(col2d.at[_Z()], cidx.at[r], isem).wait()
            pltpu.make_async_copy(row2d.at[_Z()], ridx.at[r], isem).wait()

        for k in range(nslots):
            tbl = tables[k]

            def gather_issue(j):
                pltpu.async_copy(tbl.at[ridx.at[lax.rem(j, i4)]],
                                 rows.at[pl.ds(lax.rem(j, i3) * CH, CH)], gsem)

            def gather_drain(j):
                pltpu.make_async_copy(tbl.at[ridx.at[_Z()]],
                                      rows.at[pl.ds(lax.rem(j, i3) * CH, CH)],
                                      gsem).wait()

            def scatter_issue(j):
                pltpu.async_copy(rows.at[pl.ds(lax.rem(j, i3) * CH, CH)],
                                 acc.at[cidx.at[lax.rem(j, i4)]], ssem, add=True)

            def scatter_drain():
                pltpu.make_async_copy(rows.at[pl.ds(0, CH)],
                                      acc.at[cidx.at[_Z()]], ssem).wait()

            pltpu.sync_copy(zeros_hbm.at[pl.ds(s * R_TILE, R_TILE)],
                            acc.at[pl.ds(s * R_TILE, R_TILE)])
            plsc.subcore_barrier()

            idx_issue(jnp.int32(0))
            idx_issue(jnp.int32(1))
            idx_drain(jnp.int32(0))
            gather_issue(jnp.int32(0))

            def body(j, carry):
                gather_drain(j)
                scatter_issue(j)

                @pl.when(j + 1 < CPT)
                def _():
                    idx_drain(lax.rem(j + 1, i4))

                    @pl.when(j >= 2)
                    def _():
                        scatter_drain()

                    gather_issue(j + 1)

                @pl.when(j + 2 < CPT)
                def _():
                    idx_issue(j + 2)

                return carry

            lax.fori_loop(jnp.int32(0), jnp.int32(CPT), body, jnp.int32(0))
            scatter_drain()
            scatter_drain()
            scatter_drain()
            plsc.subcore_barrier()

            t = c * nslots + k
            pltpu.sync_copy(acc.at[pl.ds(s * R_TILE, R_TILE)],
                            out_hbm.at[pl.ds(t * NP + s * R_TILE, R_TILE)])

    return prop_kernel


_deg_call = _make_deg()
_prop3 = _make_prop(3)
_prop9 = _make_prop(9)


def _dis_of(degp_blk):
    deg = (degp_blk[0] + degp_blk[1])[:, :1]
    return jnp.where(deg > 0, lax.rsqrt(jnp.where(deg > 0, deg, 1.0)), 0.0)


def _sum48(p_ref):
    sums = p_ref[0] + p_ref[1]
    return jnp.concatenate([sums[k] for k in range(K)], axis=1)


def _stage_a(x_pad, degp, wa, b1c):
    def body(x_ref, degp_ref, wa_ref, b1_ref, ta_ref, tb_ref, tc_ref, r1_ref):
        dis = _dis_of(degp_ref[...])
        h = jnp.dot(x_ref[...], wa_ref[...], precision=_HIGH,
                    preferred_element_type=jnp.float32)
        r1_ref[...] = h[:, 3 * HID:] + b1_ref[...]
        t = dis * h[:, : 3 * HID]
        for k, ref in enumerate((ta_ref, tb_ref, tc_ref)):
            ref[...] = t[:, HID * k:HID * (k + 1)]

    slot = pl.BlockSpec((BN, HID), lambda n: (n, _Z()))
    return pl.pallas_call(
        body,
        grid=(NBLK,),
        in_specs=[
            pl.BlockSpec((BN, F_IN), lambda n: (n, _Z())),
            pl.BlockSpec((2, BN, 16), lambda n: (_Z(), n, _Z())),
            pl.BlockSpec((F_IN, 6 * HID), lambda n: (_Z(), _Z())),
            pl.BlockSpec((1, 3 * HID), lambda n: (_Z(), _Z())),
        ],
        out_specs=[slot, slot, slot,
                   pl.BlockSpec((BN, 3 * HID), lambda n: (n, _Z()))],
        out_shape=[jax.ShapeDtypeStruct((NP, HID), jnp.float32)] * 3
        + [jax.ShapeDtypeStruct((NP, 3 * HID), jnp.float32)],
    )(x_pad, degp, wa, b1c)


def _stage_b(p1, degp, r1, w1bd):
    def body(p_ref, degp_ref, r1_ref, w_ref, ta_ref, tb_ref, tc_ref):
        dis = _dis_of(degp_ref[...])
        o = jax.nn.relu(dis * _sum48(p_ref) + r1_ref[...])
        t1 = dis * jnp.dot(o, w_ref[...], precision=_HIGH,
                           preferred_element_type=jnp.float32)
        for k, ref in enumerate((ta_ref, tb_ref, tc_ref)):
            ref[...] = t1[:, HID * k:HID * (k + 1)]

    slot = pl.BlockSpec((BN, HID), lambda n: (n, _Z()))
    return pl.pallas_call(
        body,
        grid=(NBLK,),
        in_specs=[
            pl.BlockSpec((NC, K, BN, HID), lambda n: (_Z(), _Z(), n, _Z())),
            pl.BlockSpec((2, BN, 16), lambda n: (_Z(), n, _Z())),
            pl.BlockSpec((BN, 3 * HID), lambda n: (n, _Z())),
            pl.BlockSpec((3 * HID, 3 * HID), lambda n: (_Z(), _Z())),
        ],
        out_specs=[slot] * 3,
        out_shape=[jax.ShapeDtypeStruct((NP, HID), jnp.float32)] * 3,
    )(p1, degp, r1, w1bd)


def _slices9(t_refs, vals, dis):
    zpad = jnp.zeros_like(vals[:, :8])
    for k in range(K):
        zk = jnp.concatenate([dis * vals[:, C_OUT * k:C_OUT * (k + 1)], zpad],
                             axis=1)
        for j in range(3):
            t_refs[3 * k + j][...] = zk[:, HID * j:HID * (j + 1)]


def _stage_c(p2, degp, r1, wc, b2c):
    def body(p_ref, degp_ref, r1_ref, wc_ref, b2_ref, *out_refs):
        dis = _dis_of(degp_ref[...])
        o = jax.nn.relu(dis * _sum48(p_ref) + r1_ref[...])
        h = (o[:, :HID] + o[:, HID:2 * HID] + o[:, 2 * HID:]) * (1.0 / 3.0)
        h = jax.nn.relu(h)
        z = jnp.dot(h, wc_ref[...], precision=_HIGH,
                    preferred_element_type=jnp.float32)
        out_refs[9][...] = z[:, 3 * C_OUT:] + b2_ref[...]
        _slices9(out_refs[:9], z[:, : 3 * C_OUT], dis)

    slot = pl.BlockSpec((BN, HID), lambda n: (n, _Z()))
    return pl.pallas_call(
        body,
        grid=(NBLK,),
        in_specs=[
            pl.BlockSpec((NC, K, BN, HID), lambda n: (_Z(), _Z(), n, _Z())),
            pl.BlockSpec((2, BN, 16), lambda n: (_Z(), n, _Z())),
            pl.BlockSpec((BN, 3 * HID), lambda n: (n, _Z())),
            pl.BlockSpec((HID, 6 * C_OUT), lambda n: (_Z(), _Z())),
            pl.BlockSpec((1, 3 * C_OUT), lambda n: (_Z(), _Z())),
        ],
        out_specs=[slot] * 9 + [pl.BlockSpec((BN, 3 * C_OUT), lambda n: (n, _Z()))],
        out_shape=[jax.ShapeDtypeStruct((NP, HID), jnp.float32)] * 9
        + [jax.ShapeDtypeStruct((NP, 3 * C_OUT), jnp.float32)],
    )(p2, degp, r1, wc, b2c)


def _cat120(p_ref, dis, r2):
    parts = []
    for k in range(K):
        s48 = jnp.concatenate(
            [p_ref[0, 3 * k + j] + p_ref[1, 3 * k + j] for j in range(3)], axis=1)
        parts.append(s48[:, :C_OUT])
    return dis * jnp.concatenate(parts, axis=1) + r2


def _stage_d(p3, degp, r2, w2bd):
    def body(p_ref, degp_ref, r2_ref, w_ref, *out_refs):
        dis = _dis_of(degp_ref[...])
        o = _cat120(p_ref[...], dis, r2_ref[...])
        t3 = jnp.dot(o, w_ref[...], precision=_HIGH,
                     preferred_element_type=jnp.float32)
        _slices9(out_refs, t3, dis)

    slot = pl.BlockSpec((BN, HID), lambda n: (n, _Z()))
    return pl.pallas_call(
        body,
        grid=(NBLK,),
        in_specs=[
            pl.BlockSpec((NC, 9, BN, HID), lambda n: (_Z(), _Z(), n, _Z())),
            pl.BlockSpec((2, BN, 16), lambda n: (_Z(), n, _Z())),
            pl.BlockSpec((BN, 3 * C_OUT), lambda n: (n, _Z())),
            pl.BlockSpec((3 * C_OUT, 3 * C_OUT), lambda n: (_Z(), _Z())),
        ],
        out_specs=[slot] * 9,
        out_shape=[jax.ShapeDtypeStruct((NP, HID), jnp.float32)] * 9,
    )(p3, degp, r2, w2bd)


def _stage_e(p4, degp, r2):
    def body(p_ref, degp_ref, r2_ref, out_ref):
        dis = _dis_of(degp_ref[...])
        o = _cat120(p_ref[...], dis, r2_ref[...])
        m = (o[:, :C_OUT] + o[:, C_OUT:2 * C_OUT] + o[:, 2 * C_OUT:]) * (1.0 / 3.0)
        mx = jnp.max(m, axis=1, keepdims=True)
        lse = jnp.log(jnp.sum(jnp.exp(m - mx), axis=1, keepdims=True)) + mx
        out_ref[...] = m - lse

    return pl.pallas_call(
        body,
        grid=(NBLK,),
        in_specs=[
            pl.BlockSpec((NC, 9, BN, HID), lambda n: (_Z(), _Z(), n, _Z())),
            pl.BlockSpec((2, BN, 16), lambda n: (_Z(), n, _Z())),
            pl.BlockSpec((BN, 3 * C_OUT), lambda n: (n, _Z())),
        ],
        out_specs=pl.BlockSpec((BN, C_OUT), lambda n: (n, _Z())),
        out_shape=jax.ShapeDtypeStruct((NP, C_OUT), jnp.float32),
    )(p4, degp, r2)


def kernel(x, edge_index, w1_init, w1, w1_root, b1, w2_init, w2, w2_root, b2):
    f32 = jnp.float32
    row = edge_index[0].astype(jnp.int32)
    col = edge_index[1].astype(jnp.int32)
    row2d = row.reshape(ECH, CH)
    col2d = col.reshape(ECH, CH)
    x_pad = jnp.pad(x.astype(f32), ((0, NP - N), (0, 0)))

    zeros16 = jnp.zeros((NP, 16), f32)
    ones16 = jnp.ones((CH_DEG, 16), f32)

    wa = jnp.concatenate(
        [jnp.concatenate([w1_init[k] for k in range(K)], axis=1),
         jnp.concatenate([w1_root[k] for k in range(K)], axis=1)], axis=1)
    b1c = b1.astype(f32).reshape(1, K * HID)
    w1bd = jax.scipy.linalg.block_diag(*[w1[k] for k in range(K)]).astype(f32)
    wc = jnp.concatenate(
        [jnp.concatenate([w2_init[k] for k in range(K)], axis=1),
         jnp.concatenate([w2_root[k] for k in range(K)], axis=1)], axis=1)
    b2c = b2.astype(f32).reshape(1, K * C_OUT)
    w2bd = jax.scipy.linalg.block_diag(*[w2[k] for k in range(K)]).astype(f32)

    degp = _deg_call(col, zeros16, ones16).reshape(NC, NP, 16)
    *t0s, r1 = _stage_a(x_pad, degp, wa, b1c)
    p1 = _prop3(row2d, col2d, zeros16, *t0s).reshape(NC, K, NP, HID)
    t1s = _stage_b(p1, degp, r1, w1bd)
    p2 = _prop3(row2d, col2d, zeros16, *t1s).reshape(NC, K, NP, HID)
    *t2s, r2 = _stage_c(p2, degp, r1, wc, b2c)
    p3 = _prop9(row2d, col2d, zeros16, *t2s).reshape(NC, 9, NP, HID)
    t3s = _stage_d(p3, degp, r2, w2bd)
    p4 = _prop9(row2d, col2d, zeros16, *t3s).reshape(NC, 9, NP, HID)
    out = _stage_e(p4, degp, r2)
    return out[:N]


# speedup vs baseline: 15.2720x; 1.0756x over previous
import functools

import jax
import jax.numpy as jnp
from jax import lax
from jax.experimental import pallas as pl
from jax.experimental.pallas import tpu as pltpu
from jax.experimental.pallas import tpu_sc as plsc

N = 100000
E = 1600000
F_IN = 128
HID = 16
C_OUT = 40
K = 3

NC, NS = 2, 16
NW = NC * NS
NP = 100096
R_TILE = NP // NS
CH = 400
ECH = E // CH
CPC = ECH // NC
CPT = CPC // NS
CZ = 782
CH_DEG = 1000
BN = 1088
NBLK = NP // BN

_HIGH = jax.lax.Precision.HIGHEST


def _Z():
    return jnp.int32(0)


def _mesh():
    return plsc.VectorSubcoreMesh(
        core_axis_name="c", subcore_axis_name="s", num_cores=NC, num_subcores=NS
    )


def _make_deg():
    etile = E // NW
    nchunks = etile // CH_DEG

    @functools.partial(
        pl.kernel,
        out_type=jax.ShapeDtypeStruct((NC * NP, 16), jnp.float32),
        mesh=_mesh(),
        scratch_types=[
            pltpu.VMEM((CH_DEG,), jnp.int32),
            pltpu.VMEM((CH_DEG, 16), jnp.float32),
            pltpu.VMEM((CZ, 16), jnp.float32),
            pltpu.VMEM_SHARED((NP, 16), jnp.float32),
        ],
        compiler_params=pltpu.CompilerParams(use_tc_tiling_on_sc=False),
    )
    def deg_kernel(col_hbm, zeros_hbm, ones_hbm, out_hbm, cidx, ones, bounce, acc):
        c = lax.axis_index("c").astype(jnp.int32)
        s = lax.axis_index("s").astype(jnp.int32)
        pltpu.sync_copy(zeros_hbm.at[pl.ds(0, CZ)], bounce)
        for j in range(R_TILE // CZ):
            pltpu.sync_copy(bounce, acc.at[pl.ds(s * R_TILE + j * CZ, CZ)])
        pltpu.sync_copy(ones_hbm, ones)
        plsc.subcore_barrier()

        def body(i, carry):
            eoff = (c * NS + s) * etile + i * CH_DEG
            pltpu.sync_copy(col_hbm.at[pl.ds(eoff, CH_DEG)], cidx)
            pltpu.sync_copy(ones, acc.at[cidx], add=True)
            return carry

        lax.fori_loop(jnp.int32(0), jnp.int32(nchunks), body, jnp.int32(0))
        plsc.subcore_barrier()
        for j in range(R_TILE // CZ):
            pltpu.sync_copy(acc.at[pl.ds(s * R_TILE + j * CZ, CZ)], bounce)
            pltpu.sync_copy(bounce,
                            out_hbm.at[pl.ds(c * NP + s * R_TILE + j * CZ, CZ)])

    return deg_kernel


def _make_prop(nslots):
    @functools.partial(
        pl.kernel,
        out_type=jax.ShapeDtypeStruct((NC * nslots * NP, 16), jnp.float32),
        mesh=_mesh(),
        scratch_types=[
            pltpu.VMEM((4, CH), jnp.int32),
            pltpu.VMEM((4, CH), jnp.int32),
            pltpu.VMEM((3 * CH, 16), jnp.float32),
            pltpu.VMEM_SHARED((NP, 16), jnp.float32),
            pltpu.SemaphoreType.DMA,
            pltpu.SemaphoreType.DMA,
            pltpu.SemaphoreType.DMA,
        ],
        compiler_params=pltpu.CompilerParams(use_tc_tiling_on_sc=False),
    )
    def prop_kernel(row2d, col2d, zeros_hbm, *refs):
        tables = refs[:nslots]
        out_hbm = refs[nslots]
        ridx, cidx, rows, acc, isem, gsem, ssem = refs[nslots + 1:]
        c = lax.axis_index("c").astype(jnp.int32)
        s = lax.axis_index("s").astype(jnp.int32)
        crow0 = c * CPC + s * CPT
        i3, i4 = jnp.int32(3), jnp.int32(4)

        def idx_issue(j):
            r = lax.rem(j, i4)
            pltpu.async_copy(col2d.at[crow0 + j], cidx.at[r], isem)
            pltpu.async_copy(row2d.at[crow0 + j], ridx.at[r], isem)

        def idx_drain(r):
            pltpu.make_async_copy(col2d.at[_Z()], cidx.at[r], isem).wait()
            pltpu.make_async_copy(row2d.at[_Z()], ridx.at[r], isem).wait()

        for k in range(nslots):
            tbl = tables[k]

            def gather_issue(j):
                pltpu.async_copy(tbl.at[ridx.at[lax.rem(j, i4)]],
                                 rows.at[pl.ds(lax.rem(j, i3) * CH, CH)], gsem)

            def gather_drain(j):
                pltpu.make_async_copy(tbl.at[ridx.at[_Z()]],
                                      rows.at[pl.ds(lax.rem(j, i3) * CH, CH)],
                                      gsem).wait()

            def scatter_issue(j):
                pltpu.async_copy(rows.at[pl.ds(lax.rem(j, i3) * CH, CH)],
                                 acc.at[cidx.at[lax.rem(j, i4)]], ssem, add=True)

            def scatter_drain():
                pltpu.make_async_copy(rows.at[pl.ds(0, CH)],
                                      acc.at[cidx.at[_Z()]], ssem).wait()

            pltpu.sync_copy(zeros_hbm.at[pl.ds(s * R_TILE, R_TILE)],
                            acc.at[pl.ds(s * R_TILE, R_TILE)])
            plsc.subcore_barrier()

            idx_issue(jnp.int32(0))
            idx_issue(jnp.int32(1))
            idx_drain(jnp.int32(0))
            gather_issue(jnp.int32(0))

            def body(j, carry):
                gather_drain(j)
                scatter_issue(j)

                @pl.when(j + 1 < CPT)
                def _():
                    idx_drain(lax.rem(j + 1, i4))

                    @pl.when(j >= 2)
                    def _():
                        scatter_drain()

                    gather_issue(j + 1)

                @pl.when(j + 2 < CPT)
                def _():
                    idx_issue(j + 2)

                return carry

            lax.fori_loop(jnp.int32(0), jnp.int32(CPT), body, jnp.int32(0))
            scatter_drain()
            scatter_drain()
            scatter_drain()
            plsc.subcore_barrier()

            t = c * nslots + k
            pltpu.sync_copy(acc.at[pl.ds(s * R_TILE, R_TILE)],
                            out_hbm.at[pl.ds(t * NP + s * R_TILE, R_TILE)])

    return prop_kernel


_deg_call = _make_deg()
_prop3 = _make_prop(3)
_prop8 = _make_prop(8)


def _dis_of(degp_blk):
    deg = (degp_blk[0] + degp_blk[1])[:, :1]
    return jnp.where(deg > 0, lax.rsqrt(jnp.where(deg > 0, deg, 1.0)), 0.0)


def _sum48(p_ref):
    sums = p_ref[0] + p_ref[1]
    return jnp.concatenate([sums[k] for k in range(K)], axis=1)


def _stage_a(x_pad, degp, wa, b1c):
    def body(x_ref, degp_ref, wa_ref, b1_ref, ta_ref, tb_ref, tc_ref, r1_ref):
        dis = _dis_of(degp_ref[...])
        h = jnp.dot(x_ref[...], wa_ref[...], precision=_HIGH,
                    preferred_element_type=jnp.float32)
        r1_ref[...] = h[:, 3 * HID:] + b1_ref[...]
        t = dis * h[:, : 3 * HID]
        for k, ref in enumerate((ta_ref, tb_ref, tc_ref)):
            ref[...] = t[:, HID * k:HID * (k + 1)]

    slot = pl.BlockSpec((BN, HID), lambda n: (n, _Z()))
    return pl.pallas_call(
        body,
        grid=(NBLK,),
        in_specs=[
            pl.BlockSpec((BN, F_IN), lambda n: (n, _Z())),
            pl.BlockSpec((2, BN, 16), lambda n: (_Z(), n, _Z())),
            pl.BlockSpec((F_IN, 6 * HID), lambda n: (_Z(), _Z())),
            pl.BlockSpec((1, 3 * HID), lambda n: (_Z(), _Z())),
        ],
        out_specs=[slot, slot, slot,
                   pl.BlockSpec((BN, 3 * HID), lambda n: (n, _Z()))],
        out_shape=[jax.ShapeDtypeStruct((NP, HID), jnp.float32)] * 3
        + [jax.ShapeDtypeStruct((NP, 3 * HID), jnp.float32)],
    )(x_pad, degp, wa, b1c)


def _stage_b(p1, degp, r1, w1bd):
    def body(p_ref, degp_ref, r1_ref, w_ref, ta_ref, tb_ref, tc_ref):
        dis = _dis_of(degp_ref[...])
        o = jax.nn.relu(dis * _sum48(p_ref) + r1_ref[...])
        t1 = dis * jnp.dot(o, w_ref[...], precision=_HIGH,
                           preferred_element_type=jnp.float32)
        for k, ref in enumerate((ta_ref, tb_ref, tc_ref)):
            ref[...] = t1[:, HID * k:HID * (k + 1)]

    slot = pl.BlockSpec((BN, HID), lambda n: (n, _Z()))
    return pl.pallas_call(
        body,
        grid=(NBLK,),
        in_specs=[
            pl.BlockSpec((NC, K, BN, HID), lambda n: (_Z(), _Z(), n, _Z())),
            pl.BlockSpec((2, BN, 16), lambda n: (_Z(), n, _Z())),
            pl.BlockSpec((BN, 3 * HID), lambda n: (n, _Z())),
            pl.BlockSpec((3 * HID, 3 * HID), lambda n: (_Z(), _Z())),
        ],
        out_specs=[slot] * 3,
        out_shape=[jax.ShapeDtypeStruct((NP, HID), jnp.float32)] * 3,
    )(p1, degp, r1, w1bd)


def _slices8(t_refs, vals, dis):
    v = dis * vals
    for k in range(K):
        t_refs[2 * k][...] = v[:, C_OUT * k:C_OUT * k + 16]
        t_refs[2 * k + 1][...] = v[:, C_OUT * k + 16:C_OUT * k + 32]
    t_refs[6][...] = jnp.concatenate([v[:, 32:40], v[:, 72:80]], axis=1)
    t_refs[7][...] = jnp.concatenate([v[:, 112:120],
                                      jnp.zeros_like(v[:, :8])], axis=1)


def _stage_c(p2, degp, r1, wc, b2c):
    def body(p_ref, degp_ref, r1_ref, wc_ref, b2_ref, *out_refs):
        dis = _dis_of(degp_ref[...])
        o = jax.nn.relu(dis * _sum48(p_ref) + r1_ref[...])
        h = (o[:, :HID] + o[:, HID:2 * HID] + o[:, 2 * HID:]) * (1.0 / 3.0)
        h = jax.nn.relu(h)
        z = jnp.dot(h, wc_ref[...], precision=_HIGH,
                    preferred_element_type=jnp.float32)
        out_refs[8][...] = z[:, 3 * C_OUT:] + b2_ref[...]
        _slices8(out_refs[:8], z[:, : 3 * C_OUT], dis)

    slot = pl.BlockSpec((BN, HID), lambda n: (n, _Z()))
    return pl.pallas_call(
        body,
        grid=(NBLK,),
        in_specs=[
            pl.BlockSpec((NC, K, BN, HID), lambda n: (_Z(), _Z(), n, _Z())),
            pl.BlockSpec((2, BN, 16), lambda n: (_Z(), n, _Z())),
            pl.BlockSpec((BN, 3 * HID), lambda n: (n, _Z())),
            pl.BlockSpec((HID, 6 * C_OUT), lambda n: (_Z(), _Z())),
            pl.BlockSpec((1, 3 * C_OUT), lambda n: (_Z(), _Z())),
        ],
        out_specs=[slot] * 8 + [pl.BlockSpec((BN, 3 * C_OUT), lambda n: (n, _Z()))],
        out_shape=[jax.ShapeDtypeStruct((NP, HID), jnp.float32)] * 8
        + [jax.ShapeDtypeStruct((NP, 3 * C_OUT), jnp.float32)],
    )(p2, degp, r1, wc, b2c)


def _cat120(p_ref, dis, r2):
    def q(i):
        return p_ref[0, i] + p_ref[1, i]
    t = q(6)
    t2 = q(7)
    parts = [q(0), q(1), t[:, :8], q(2), q(3), t[:, 8:],
             q(4), q(5), t2[:, :8]]
    return dis * jnp.concatenate(parts, axis=1) + r2


def _stage_d(p3, degp, r2, w2bd):
    def body(p_ref, degp_ref, r2_ref, w_ref, *out_refs):
        dis = _dis_of(degp_ref[...])
        o = _cat120(p_ref[...], dis, r2_ref[...])
        t3 = jnp.dot(o, w_ref[...], precision=_HIGH,
                     preferred_element_type=jnp.float32)
        _slices8(out_refs, t3, dis)

    slot = pl.BlockSpec((BN, HID), lambda n: (n, _Z()))
    return pl.pallas_call(
        body,
        grid=(NBLK,),
        in_specs=[
            pl.BlockSpec((NC, 8, BN, HID), lambda n: (_Z(), _Z(), n, _Z())),
            pl.BlockSpec((2, BN, 16), lambda n: (_Z(), n, _Z())),
            pl.BlockSpec((BN, 3 * C_OUT), lambda n: (n, _Z())),
            pl.BlockSpec((3 * C_OUT, 3 * C_OUT), lambda n: (_Z(), _Z())),
        ],
        out_specs=[slot] * 8,
        out_shape=[jax.ShapeDtypeStruct((NP, HID), jnp.float32)] * 8,
    )(p3, degp, r2, w2bd)


def _stage_e(p4, degp, r2):
    def body(p_ref, degp_ref, r2_ref, out_ref):
        dis = _dis_of(degp_ref[...])
        o = _cat120(p_ref[...], dis, r2_ref[...])
        m = (o[:, :C_OUT] + o[:, C_OUT:2 * C_OUT] + o[:, 2 * C_OUT:]) * (1.0 / 3.0)
        mx = jnp.max(m, axis=1, keepdims=True)
        lse = jnp.log(jnp.sum(jnp.exp(m - mx), axis=1, keepdims=True)) + mx
        out_ref[...] = m - lse

    return pl.pallas_call(
        body,
        grid=(NBLK,),
        in_specs=[
            pl.BlockSpec((NC, 8, BN, HID), lambda n: (_Z(), _Z(), n, _Z())),
            pl.BlockSpec((2, BN, 16), lambda n: (_Z(), n, _Z())),
            pl.BlockSpec((BN, 3 * C_OUT), lambda n: (n, _Z())),
        ],
        out_specs=pl.BlockSpec((BN, C_OUT), lambda n: (n, _Z())),
        out_shape=jax.ShapeDtypeStruct((NP, C_OUT), jnp.float32),
    )(p4, degp, r2)


def kernel(x, edge_index, w1_init, w1, w1_root, b1, w2_init, w2, w2_root, b2):
    f32 = jnp.float32
    row = edge_index[0].astype(jnp.int32)
    col = edge_index[1].astype(jnp.int32)
    row2d = row.reshape(ECH, CH)
    col2d = col.reshape(ECH, CH)
    x_pad = jnp.pad(x.astype(f32), ((0, NP - N), (0, 0)))

    zeros16 = jnp.zeros((NP, 16), f32)
    ones16 = jnp.ones((CH_DEG, 16), f32)

    wa = jnp.concatenate(
        [jnp.concatenate([w1_init[k] for k in range(K)], axis=1),
         jnp.concatenate([w1_root[k] for k in range(K)], axis=1)], axis=1)
    b1c = b1.astype(f32).reshape(1, K * HID)
    w1bd = jax.scipy.linalg.block_diag(*[w1[k] for k in range(K)]).astype(f32)
    wc = jnp.concatenate(
        [jnp.concatenate([w2_init[k] for k in range(K)], axis=1),
         jnp.concatenate([w2_root[k] for k in range(K)], axis=1)], axis=1)
    b2c = b2.astype(f32).reshape(1, K * C_OUT)
    w2bd = jax.scipy.linalg.block_diag(*[w2[k] for k in range(K)]).astype(f32)

    degp = _deg_call(col, zeros16, ones16).reshape(NC, NP, 16)
    *t0s, r1 = _stage_a(x_pad, degp, wa, b1c)
    p1 = _prop3(row2d, col2d, zeros16, *t0s).reshape(NC, K, NP, HID)
    t1s = _stage_b(p1, degp, r1, w1bd)
    p2 = _prop3(row2d, col2d, zeros16, *t1s).reshape(NC, K, NP, HID)
    *t2s, r2 = _stage_c(p2, degp, r1, wc, b2c)
    p3 = _prop8(row2d, col2d, zeros16, *t2s).reshape(NC, 8, NP, HID)
    t3s = _stage_d(p3, degp, r2, w2bd)
    p4 = _prop8(row2d, col2d, zeros16, *t3s).reshape(NC, 8, NP, HID)
    out = _stage_e(p4, degp, r2)
    return out[:N]
